# baseline (speedup 1.0000x reference)
import functools

import jax
import jax.numpy as jnp
from jax import lax
from jax.experimental import pallas as pl
from jax.experimental.pallas import tpu as pltpu
from jax.experimental.pallas import tpu_sc as plsc

HID = 64
NC = 2
NS = 16
NW = NC * NS
IDX_W = 128
NBUF = 4
LANES = 16
TPITCH = IDX_W


SPLIT = 50048


def _proj_body(e1_ref, e2_ref, w_ref, b_ref, o_ref):
    def half(e_ref):
        return (
            lax.dot_general(
                e_ref[...], w_ref[...],
                dimension_numbers=(((0,), (0,)), ((), ())),
                preferred_element_type=jnp.float32,
            )
            + b_ref[...]
        )

    o_ref[:, 0:HID] = half(e1_ref)
    o_ref[:, HID:2 * HID] = half(e2_ref)


def _project_table(emb_t, w_t, b2):
    BLK = 2176
    grid = SPLIT // BLK
    return pl.pallas_call(
        _proj_body,
        grid=(grid,),
        in_specs=[
            pl.BlockSpec((HID, BLK), lambda i: (0, i)),
            pl.BlockSpec((HID, BLK), lambda i, g=grid: (0, i + g)),
            pl.BlockSpec((HID, HID), lambda i: (0, 0)),
            pl.BlockSpec((1, HID), lambda i: (0, 0)),
        ],
        out_specs=pl.BlockSpec((BLK, 2 * HID), lambda i: (i, 0)),
        out_shape=jax.ShapeDtypeStruct((SPLIT, 2 * HID), jnp.float32),
    )(emb_t, emb_t, w_t, b2)


def _make_sc_gather(B, L, V):
    n_rows_total = B * L // IDX_W
    rows_per_w = n_rows_total // NW
    blocks_per_l = B // IDX_W
    n_steps = rows_per_w // NBUF
    FH = HID // 8
    mesh = plsc.VectorSubcoreMesh(
        core_axis_name="c", subcore_axis_name="s",
        num_cores=NC, num_subcores=NS,
    )

    @functools.partial(
        pl.kernel,
        out_type=jax.ShapeDtypeStruct((L * HID * B // IDX_W, IDX_W),
                                      jnp.float32),
        mesh=mesh,
        scratch_types=[
            pltpu.VMEM((rows_per_w, IDX_W), jnp.int32),
            [pltpu.VMEM((IDX_W, HID), jnp.float32) for _ in range(NBUF)],
            [pltpu.VMEM((HID, TPITCH), jnp.float32) for _ in range(2)],
            [pltpu.SemaphoreType.DMA for _ in range(NBUF)],
            [pltpu.SemaphoreType.DMA for _ in range(2)],
        ],
        compiler_params=pltpu.CompilerParams(
            needs_layout_passes=False, use_tc_tiling_on_sc=False),
    )
    def gather_kernel(table_hbm, idx_hbm, out_hbm, idx_v, rows, tbuf,
                      sem_g, sem_s):
        wid = lax.axis_index("s") * NC + lax.axis_index("c")
        row0 = pl.multiple_of(wid * rows_per_w, rows_per_w)
        pltpu.sync_copy(idx_hbm.at[pl.ds(row0, rows_per_w)], idx_v)

        @plsc.parallel_loop(0, rows_per_w * (IDX_W // LANES), unroll=8)
        def _remap(i):
            r = i // (IDX_W // LANES)
            m = i % (IDX_W // LANES)
            v = idx_v[r, pl.ds(m * LANES, LANES)]
            v = 2 * v - jnp.where(v >= SPLIT, 2 * SPLIT - 1, 0)
            idx_v[r, pl.ds(m * LANES, LANES)] = v

        def fire_gather(r, slot):
            return pltpu.async_copy(
                table_hbm.at[idx_v.at[r]], rows[slot], sem_g[slot])

        def store_tile(half, r, wait_only=False):
            g = row0 + r
            l = g // blocks_per_l
            bc = g % blocks_per_l
            for fh in range(FH):
                orow = pl.multiple_of(((l * FH + fh) * blocks_per_l + bc) * 8,
                                      8)
                src = tbuf[half].at[pl.ds(fh * 8, 8), pl.ds(0, IDX_W)]
                dst = out_hbm.at[pl.ds(orow, 8)]
                if wait_only:
                    pltpu.make_async_copy(src, dst, sem_s[half]).wait()
                else:
                    pltpu.async_copy(src, dst, sem_s[half])

        for k in range(NBUF - 1):
            fire_gather(k, k)

        row_idx = [lax.iota(jnp.int32, LANES) + 16 * kk for kk in range(4)]

        def transpose_row(slot, half):
            src = rows[slot]
            dst = tbuf[half]

            @plsc.parallel_loop(0, IDX_W, unroll=8)
            def tbody(t):
                col = jnp.full((LANES,), t, jnp.int32)
                for kk in range(HID // LANES):
                    v = src[t, pl.ds(LANES * kk, LANES)]
                    plsc.store_scatter(dst, [row_idx[kk], col], v)

        def body(s, carry):
            for k in range(NBUF):
                r = s * NBUF + k
                half = k % 2
                pltpu.make_async_copy(
                    table_hbm.at[idx_v.at[r]], rows[k], sem_g[k]).wait()

                def _wait_store():
                    store_tile(half, r, wait_only=True)
                if k >= 2:
                    _wait_store()
                else:
                    pl.when(s > 0)(_wait_store)

                transpose_row(k, half)
                store_tile(half, r)

                nxt = r + NBUF - 1
                slot = (k + NBUF - 1) % NBUF

                def _fire():
                    fire_gather(nxt, slot)
                if k == 0:
                    _fire()
                else:
                    pl.when(nxt < rows_per_w)(_fire)
            return carry

        lax.fori_loop(0, n_steps, body, 0)
        for half in range(2):
            store_tile(half, rows_per_w - 2 + half, wait_only=True)

    return gather_kernel


def kernel(token_ids, emb_table, W, b):
    B, L = token_ids.shape
    V = emb_table.shape[0]
    proj2 = _project_table(jnp.transpose(emb_table), W.T, b.reshape(1, HID))
    table = proj2.reshape(2 * SPLIT, HID)
    idx2 = jnp.transpose(token_ids).reshape(B * L // IDX_W, IDX_W)
    idx2 = idx2.astype(jnp.int32)
    out2 = _make_sc_gather(B, L, V)(table, idx2)
    t5 = out2.reshape(L, HID // 8, B // IDX_W, 8, IDX_W)
    return jnp.transpose(t5, (2, 4, 0, 1, 3)).reshape(B, L, HID)


# speedup vs baseline: 3.7069x; 3.7069x over previous
import functools

import jax
import jax.numpy as jnp
from jax import lax
from jax.experimental import pallas as pl
from jax.experimental.pallas import tpu as pltpu
from jax.experimental.pallas import tpu_sc as plsc

HID = 64
NC = 2
NS = 16
NW = NC * NS
IDX_W = 128
NBUF = 4
LANES = 16
TPITCH = IDX_W + 1


SPLIT = 50048


def _proj_body(e1_ref, e2_ref, w_ref, b_ref, o_ref):
    def half(e_ref):
        return (
            lax.dot_general(
                e_ref[...], w_ref[...],
                dimension_numbers=(((0,), (0,)), ((), ())),
                preferred_element_type=jnp.float32,
            )
            + b_ref[...]
        )

    o_ref[:, 0:HID] = half(e1_ref)
    o_ref[:, HID:2 * HID] = half(e2_ref)


def _project_table(emb_t, w_t, b2):
    BLK = 2176
    grid = SPLIT // BLK
    return pl.pallas_call(
        _proj_body,
        grid=(grid,),
        in_specs=[
            pl.BlockSpec((HID, BLK), lambda i: (0, i)),
            pl.BlockSpec((HID, BLK), lambda i, g=grid: (0, i + g)),
            pl.BlockSpec((HID, HID), lambda i: (0, 0)),
            pl.BlockSpec((1, HID), lambda i: (0, 0)),
        ],
        out_specs=pl.BlockSpec((BLK, 2 * HID), lambda i: (i, 0)),
        out_shape=jax.ShapeDtypeStruct((SPLIT, 2 * HID), jnp.float32),
    )(emb_t, emb_t, w_t, b2)


def _make_sc_gather(B, L, V):
    n_rows_total = B * L // IDX_W
    rows_per_w = n_rows_total // NW
    blocks_per_l = B // IDX_W
    n_steps = rows_per_w // NBUF
    FH = HID // 8
    mesh = plsc.VectorSubcoreMesh(
        core_axis_name="c", subcore_axis_name="s",
        num_cores=NC, num_subcores=NS,
    )

    @functools.partial(
        pl.kernel,
        out_type=jax.ShapeDtypeStruct((L * HID * B // IDX_W, IDX_W),
                                      jnp.float32),
        mesh=mesh,
        scratch_types=[
            pltpu.VMEM((rows_per_w, IDX_W), jnp.int32),
            [pltpu.VMEM((IDX_W, HID), jnp.float32) for _ in range(NBUF)],
            [pltpu.VMEM((HID, TPITCH), jnp.float32) for _ in range(2)],
            [pltpu.SemaphoreType.DMA for _ in range(NBUF)],
            [pltpu.SemaphoreType.DMA for _ in range(2)],
        ],
        compiler_params=pltpu.CompilerParams(
            needs_layout_passes=False, use_tc_tiling_on_sc=False),
    )
    def gather_kernel(table_hbm, idx_hbm, out_hbm, idx_v, rows, tbuf,
                      sem_g, sem_s):
        wid = lax.axis_index("s") * NC + lax.axis_index("c")
        row0 = pl.multiple_of(wid * rows_per_w, rows_per_w)
        pltpu.sync_copy(idx_hbm.at[pl.ds(row0, rows_per_w)], idx_v)

        @plsc.parallel_loop(0, rows_per_w * (IDX_W // LANES), unroll=8)
        def _remap(i):
            r = i // (IDX_W // LANES)
            m = i % (IDX_W // LANES)
            v = idx_v[r, pl.ds(m * LANES, LANES)]
            v = 2 * v - jnp.where(v >= SPLIT, 2 * SPLIT - 1, 0)
            idx_v[r, pl.ds(m * LANES, LANES)] = v

        def fire_gather(r, slot):
            return pltpu.async_copy(
                table_hbm.at[idx_v.at[r]], rows[slot], sem_g[slot])

        def store_tile(half, r, wait_only=False):
            g = row0 + r
            l = g // blocks_per_l
            bc = g % blocks_per_l
            for fh in range(FH):
                orow = pl.multiple_of(((l * FH + fh) * blocks_per_l + bc) * 8,
                                      8)
                src = tbuf[half].at[pl.ds(fh * 8, 8), pl.ds(0, IDX_W)]
                dst = out_hbm.at[pl.ds(orow, 8)]
                if wait_only:
                    pltpu.make_async_copy(src, dst, sem_s[half]).wait()
                else:
                    pltpu.async_copy(src, dst, sem_s[half])

        for k in range(NBUF - 1):
            fire_gather(k, k)

        row_idx = [lax.iota(jnp.int32, LANES) + 16 * kk for kk in range(4)]

        def transpose_row(slot, half):
            src = rows[slot]
            dst = tbuf[half]

            @plsc.parallel_loop(0, IDX_W, unroll=8)
            def tbody(t):
                col = jnp.full((LANES,), t, jnp.int32)
                for kk in range(HID // LANES):
                    v = src[t, pl.ds(LANES * kk, LANES)]
                    plsc.store_scatter(dst, [row_idx[kk], col], v)

        def body(s, carry):
            for k in range(NBUF):
                r = s * NBUF + k
                half = k % 2
                pltpu.make_async_copy(
                    table_hbm.at[idx_v.at[r]], rows[k], sem_g[k]).wait()

                def _wait_store():
                    store_tile(half, r, wait_only=True)
                if k >= 2:
                    _wait_store()
                else:
                    pl.when(s > 0)(_wait_store)

                transpose_row(k, half)
                store_tile(half, r)

                nxt = r + NBUF - 1
                slot = (k + NBUF - 1) % NBUF

                def _fire():
                    fire_gather(nxt, slot)
                if k == 0:
                    _fire()
                else:
                    pl.when(nxt < rows_per_w)(_fire)
            return carry

        lax.fori_loop(0, n_steps, body, 0)
        for half in range(2):
            store_tile(half, rows_per_w - 2 + half, wait_only=True)

    return gather_kernel


def kernel(token_ids, emb_table, W, b):
    B, L = token_ids.shape
    V = emb_table.shape[0]
    proj2 = _project_table(jnp.transpose(emb_table), W.T, b.reshape(1, HID))
    table = proj2.reshape(2 * SPLIT, HID)
    idx2 = jnp.transpose(token_ids).reshape(B * L // IDX_W, IDX_W)
    idx2 = idx2.astype(jnp.int32)
    out2 = _make_sc_gather(B, L, V)(table, idx2)
    t5 = out2.reshape(L, HID // 8, B // IDX_W, 8, IDX_W)
    return jnp.transpose(t5, (2, 4, 0, 1, 3)).reshape(B, L, HID)


# speedup vs baseline: 3.7446x; 1.0102x over previous
import functools

import jax
import jax.numpy as jnp
from jax import lax
from jax.experimental import pallas as pl
from jax.experimental.pallas import tpu as pltpu
from jax.experimental.pallas import tpu_sc as plsc

HID = 64
NC = 2
NS = 16
NW = NC * NS
IDX_W = 128
NBUF = 8
LANES = 16
TPITCH = IDX_W + 1


SPLIT = 50048


def _proj_body(e1_ref, e2_ref, w_ref, b_ref, o_ref):
    def half(e_ref):
        return (
            lax.dot_general(
                e_ref[...], w_ref[...],
                dimension_numbers=(((0,), (0,)), ((), ())),
                preferred_element_type=jnp.float32,
            )
            + b_ref[...]
        )

    o_ref[:, 0:HID] = half(e1_ref)
    o_ref[:, HID:2 * HID] = half(e2_ref)


def _project_table(emb_t, w_t, b2):
    BLK = 2176
    grid = SPLIT // BLK
    return pl.pallas_call(
        _proj_body,
        grid=(grid,),
        in_specs=[
            pl.BlockSpec((HID, BLK), lambda i: (0, i)),
            pl.BlockSpec((HID, BLK), lambda i, g=grid: (0, i + g)),
            pl.BlockSpec((HID, HID), lambda i: (0, 0)),
            pl.BlockSpec((1, HID), lambda i: (0, 0)),
        ],
        out_specs=pl.BlockSpec((BLK, 2 * HID), lambda i: (i, 0)),
        out_shape=jax.ShapeDtypeStruct((SPLIT, 2 * HID), jnp.float32),
    )(emb_t, emb_t, w_t, b2)


def _make_sc_gather(B, L, V):
    n_rows_total = B * L // IDX_W
    rows_per_w = n_rows_total // NW
    blocks_per_l = B // IDX_W
    n_steps = rows_per_w // NBUF
    FH = HID // 8
    mesh = plsc.VectorSubcoreMesh(
        core_axis_name="c", subcore_axis_name="s",
        num_cores=NC, num_subcores=NS,
    )

    @functools.partial(
        pl.kernel,
        out_type=jax.ShapeDtypeStruct((L * HID * B // IDX_W, IDX_W),
                                      jnp.float32),
        mesh=mesh,
        scratch_types=[
            pltpu.VMEM((rows_per_w, IDX_W), jnp.int32),
            [pltpu.VMEM((IDX_W, HID), jnp.float32) for _ in range(NBUF)],
            [pltpu.VMEM((HID, TPITCH), jnp.float32) for _ in range(2)],
            [pltpu.SemaphoreType.DMA for _ in range(NBUF)],
            [pltpu.SemaphoreType.DMA for _ in range(2)],
        ],
        compiler_params=pltpu.CompilerParams(
            needs_layout_passes=False, use_tc_tiling_on_sc=False),
    )
    def gather_kernel(table_hbm, idx_hbm, out_hbm, idx_v, rows, tbuf,
                      sem_g, sem_s):
        wid = lax.axis_index("s") * NC + lax.axis_index("c")
        row0 = pl.multiple_of(wid * rows_per_w, rows_per_w)
        pltpu.sync_copy(idx_hbm.at[pl.ds(row0, rows_per_w)], idx_v)

        @plsc.parallel_loop(0, rows_per_w * (IDX_W // LANES), unroll=8)
        def _remap(i):
            r = i // (IDX_W // LANES)
            m = i % (IDX_W // LANES)
            v = idx_v[r, pl.ds(m * LANES, LANES)]
            v = 2 * v - jnp.where(v >= SPLIT, 2 * SPLIT - 1, 0)
            idx_v[r, pl.ds(m * LANES, LANES)] = v

        def fire_gather(r, slot):
            return pltpu.async_copy(
                table_hbm.at[idx_v.at[r]], rows[slot], sem_g[slot])

        def store_tile(half, r, wait_only=False):
            g = row0 + r
            l = g // blocks_per_l
            bc = g % blocks_per_l
            for fh in range(FH):
                orow = pl.multiple_of(((l * FH + fh) * blocks_per_l + bc) * 8,
                                      8)
                src = tbuf[half].at[pl.ds(fh * 8, 8), pl.ds(0, IDX_W)]
                dst = out_hbm.at[pl.ds(orow, 8)]
                if wait_only:
                    pltpu.make_async_copy(src, dst, sem_s[half]).wait()
                else:
                    pltpu.async_copy(src, dst, sem_s[half])

        for k in range(NBUF - 1):
            fire_gather(k, k)

        row_idx = [lax.iota(jnp.int32, LANES) + 16 * kk for kk in range(4)]

        def transpose_row(slot, half):
            src = rows[slot]
            dst = tbuf[half]

            @plsc.parallel_loop(0, IDX_W, unroll=8)
            def tbody(t):
                col = jnp.full((LANES,), t, jnp.int32)
                for kk in range(HID // LANES):
                    v = src[t, pl.ds(LANES * kk, LANES)]
                    plsc.store_scatter(dst, [row_idx[kk], col], v)

        def body(s, carry):
            for k in range(NBUF):
                r = s * NBUF + k
                half = k % 2
                pltpu.make_async_copy(
                    table_hbm.at[idx_v.at[r]], rows[k], sem_g[k]).wait()

                def _wait_store():
                    store_tile(half, r, wait_only=True)
                if k >= 2:
                    _wait_store()
                else:
                    pl.when(s > 0)(_wait_store)

                transpose_row(k, half)
                store_tile(half, r)

                nxt = r + NBUF - 1
                slot = (k + NBUF - 1) % NBUF

                def _fire():
                    fire_gather(nxt, slot)
                if k == 0:
                    _fire()
                else:
                    pl.when(nxt < rows_per_w)(_fire)
            return carry

        lax.fori_loop(0, n_steps, body, 0)
        for half in range(2):
            store_tile(half, rows_per_w - 2 + half, wait_only=True)

    return gather_kernel


def kernel(token_ids, emb_table, W, b):
    B, L = token_ids.shape
    V = emb_table.shape[0]
    proj2 = _project_table(jnp.transpose(emb_table), W.T, b.reshape(1, HID))
    table = proj2.reshape(2 * SPLIT, HID)
    idx2 = jnp.transpose(token_ids).reshape(B * L // IDX_W, IDX_W)
    idx2 = idx2.astype(jnp.int32)
    out2 = _make_sc_gather(B, L, V)(table, idx2)
    t5 = out2.reshape(L, HID // 8, B // IDX_W, 8, IDX_W)
    return jnp.transpose(t5, (2, 4, 0, 1, 3)).reshape(B, L, HID)
